# Initial kernel scaffold; baseline (speedup 1.0000x reference)
import functools

import jax
import jax.numpy as jnp
from jax import lax
from jax.experimental import pallas as pl
from jax.experimental.pallas import tpu as pltpu
from jax.experimental.pallas import tpu_sc as plsc

N_NODES = 10000
DIM = 128
N_EDGES = 320000

NC = 2
NS = 16
NW = NC * NS
EPW = N_EDGES // NW
K = 80
NCHUNK = EPW // K
RPT = N_NODES // NS
DEGW = 16

_MESH = plsc.VectorSubcoreMesh(core_axis_name="c", subcore_axis_name="s")


def _deg_body(dst_hbm, ones_hbm, zeros_hbm, deg_out, deg_sh, didx_v, ones_v):
    c = lax.axis_index("c")
    s = lax.axis_index("s")
    wid = s * NC + c
    pltpu.sync_copy(zeros_hbm.at[pl.ds(s * RPT, RPT)],
                    deg_sh.at[pl.ds(s * RPT, RPT)])
    pltpu.sync_copy(ones_hbm, ones_v)
    pltpu.sync_copy(dst_hbm.at[wid], didx_v)
    plsc.subcore_barrier()

    @pl.loop(0, NCHUNK)
    def _chunk(j):
        pltpu.sync_copy(ones_v, deg_sh.at[didx_v.at[j]], add=True)

    plsc.subcore_barrier()
    pltpu.sync_copy(deg_sh.at[pl.ds(s * RPT, RPT)],
                    deg_out.at[c, pl.ds(s * RPT, RPT)])


def _degree_hist(dst3, ones, zeros):
    return pl.kernel(
        _deg_body,
        out_type=jax.ShapeDtypeStruct((NC, N_NODES, DEGW), jnp.float32),
        mesh=_MESH,
        scratch_types=[
            pltpu.VMEM_SHARED((N_NODES, DEGW), jnp.float32),
            pltpu.VMEM((NCHUNK, K), jnp.int32),
            pltpu.VMEM((K, DEGW), jnp.float32),
        ],
    )(dst3, ones, zeros)


def _edge_body(src_hbm, dst_hbm, y_hbm, zeros_hbm, acc_out,
               acc_sh, sidx_v, didx_v, rows_v, sem):
    c = lax.axis_index("c")
    s = lax.axis_index("s")
    wid = s * NC + c
    pltpu.sync_copy(zeros_hbm.at[pl.ds(s * RPT, RPT)],
                    acc_sh.at[pl.ds(s * RPT, RPT)])
    pltpu.sync_copy(src_hbm.at[wid], sidx_v)
    pltpu.sync_copy(dst_hbm.at[wid], didx_v)
    plsc.subcore_barrier()

    @pl.loop(0, NCHUNK)
    def _chunk(j):
        pltpu.async_copy(y_hbm.at[sidx_v.at[j]], rows_v, sem).wait()
        pltpu.sync_copy(rows_v, acc_sh.at[didx_v.at[j]], add=True)

    plsc.subcore_barrier()
    pltpu.sync_copy(acc_sh.at[pl.ds(s * RPT, RPT)],
                    acc_out.at[c, pl.ds(s * RPT, RPT)])


def _edge_pass(src3, dst3, y, zeros):
    return pl.kernel(
        _edge_body,
        out_type=jax.ShapeDtypeStruct((NC, N_NODES, DIM), jnp.float32),
        mesh=_MESH,
        scratch_types=[
            pltpu.VMEM_SHARED((N_NODES, DIM), jnp.float32),
            pltpu.VMEM((NCHUNK, K), jnp.int32),
            pltpu.VMEM((NCHUNK, K), jnp.int32),
            pltpu.VMEM((K, DIM), jnp.float32),
            pltpu.SemaphoreType.DMA,
        ],
    )(src3, dst3, y, zeros)


RB = 2000


def _scale_body(deg_ref, emb_ref, w_ref, y_ref, dinv_ref):
    deg = deg_ref[0, :, 0:1] + deg_ref[1, :, 0:1] + 1.0
    dinv = lax.rsqrt(jnp.maximum(deg, 1.0))
    xw = jnp.dot(emb_ref[...], w_ref[...], preferred_element_type=jnp.float32)
    y_ref[...] = xw * dinv
    dinv_ref[...] = dinv


def _scale(dega, emb, w):
    grid = N_NODES // RB
    return pl.pallas_call(
        _scale_body,
        grid=(grid,),
        in_specs=[
            pl.BlockSpec((NC, RB, DEGW), lambda i: (0, i, 0)),
            pl.BlockSpec((RB, DIM), lambda i: (i, 0)),
            pl.BlockSpec((DIM, DIM), lambda i: (0, 0)),
        ],
        out_specs=[
            pl.BlockSpec((RB, DIM), lambda i: (i, 0)),
            pl.BlockSpec((RB, 1), lambda i: (i, 0)),
        ],
        out_shape=[
            jax.ShapeDtypeStruct((N_NODES, DIM), jnp.float32),
            jax.ShapeDtypeStruct((N_NODES, 1), jnp.float32),
        ],
    )(dega, emb, w)


def _final_body(acc_ref, y_ref, dinv_ref, b_ref, out_ref):
    tot = acc_ref[0] + acc_ref[1] + y_ref[...]
    out_ref[...] = dinv_ref[...] * tot + b_ref[...]


def _final(acc, y, dinv, b2):
    grid = N_NODES // RB
    return pl.pallas_call(
        _final_body,
        grid=(grid,),
        in_specs=[
            pl.BlockSpec((NC, RB, DIM), lambda i: (0, i, 0)),
            pl.BlockSpec((RB, DIM), lambda i: (i, 0)),
            pl.BlockSpec((RB, 1), lambda i: (i, 0)),
            pl.BlockSpec((1, DIM), lambda i: (0, 0)),
        ],
        out_specs=pl.BlockSpec((RB, DIM), lambda i: (i, 0)),
        out_shape=jax.ShapeDtypeStruct((N_NODES, DIM), jnp.float32),
    )(acc, y, dinv, b2)


@jax.jit
def kernel(edge_index, emb, W, b):
    ei = jnp.asarray(edge_index, jnp.int32)
    src3 = ei[0].reshape(NW, NCHUNK, K)
    dst3 = ei[1].reshape(NW, NCHUNK, K)
    ones = jnp.ones((K, DEGW), jnp.float32)
    zeros = jnp.zeros((N_NODES, DIM), jnp.float32)

    dega = _degree_hist(dst3, ones, zeros[:, :DEGW])
    y, dinv = _scale(dega, emb, W)
    acc = _edge_pass(src3, dst3, y, zeros)
    out = _final(acc, y, dinv, b.reshape(1, DIM))
    return out


# speedup vs baseline: 27.7633x; 27.7633x over previous
import jax
import jax.numpy as jnp
from jax import lax
from jax.experimental import pallas as pl
from jax.experimental.pallas import tpu as pltpu
from jax.experimental.pallas import tpu_sc as plsc

N_NODES = 10000
DIM = 128
N_EDGES = 320000

NC = 2
NS = 16
NW = NC * NS
EPW = N_EDGES // NW
K = 80
NCHUNK = EPW // K
N_PAD = 10240
RPT = N_PAD // NS
DEGW = 16

_MESH = plsc.VectorSubcoreMesh(core_axis_name="c", subcore_axis_name="s")


def _deg_body(dst_hbm, deg_out, slots_sh, hist_v, dall_v, tmp_v, col_v):
    c = lax.axis_index("c")
    s = lax.axis_index("s")
    wid = s * NC + c
    pltpu.sync_copy(dst_hbm.at[pl.ds(wid * EPW, EPW)], dall_v)

    @pl.loop(0, N_PAD // 16)
    def _z(i):
        hist_v[pl.ds(i * 16, 16)] = jnp.zeros((16,), jnp.float32)

    ones16 = jnp.ones((16,), jnp.float32)

    @pl.loop(0, EPW // 16)
    def _h(i):
        idx = dall_v[pl.ds(i * 16, 16)]
        plsc.addupdate_scatter(hist_v, [idx], ones16)

    pltpu.sync_copy(hist_v, slots_sh.at[pl.ds(s * N_PAD, N_PAD)])
    plsc.subcore_barrier()

    @pl.loop(0, N_PAD // 16)
    def _z2(i):
        hist_v[pl.ds(i * 16, 16)] = jnp.zeros((16,), jnp.float32)

    for t in range(NS):
        pltpu.sync_copy(slots_sh.at[pl.ds(t * N_PAD + s * RPT, RPT)], tmp_v)

        @pl.loop(0, RPT // 16)
        def _acc(i):
            sl = pl.ds(i * 16, 16)
            hist_v[sl] = hist_v[sl] + tmp_v[sl]

    zcol = jnp.zeros((16,), jnp.int32)

    @pl.loop(0, RPT // 16)
    def _col(i):
        rows = jax.lax.iota(jnp.int32, 16) + i * 16
        vals = hist_v[pl.ds(i * 16, 16)]
        plsc.store_scatter(col_v, [rows, zcol], vals)

    pltpu.sync_copy(col_v, deg_out.at[c, pl.ds(s * RPT, RPT)])


def _degree_hist(dst):
    return pl.kernel(
        _deg_body,
        out_type=jax.ShapeDtypeStruct((NC, N_PAD, DEGW), jnp.float32),
        mesh=_MESH,
        compiler_params=pltpu.CompilerParams(needs_layout_passes=False),
        scratch_types=[
            pltpu.VMEM_SHARED((NS * N_PAD,), jnp.float32),
            pltpu.VMEM((N_PAD,), jnp.float32),
            pltpu.VMEM((EPW,), jnp.int32),
            pltpu.VMEM((RPT,), jnp.float32),
            pltpu.VMEM((RPT, DEGW), jnp.float32),
        ],
    )(dst)


def _edge_body(src_hbm, dst_hbm, y_hbm, zeros_hbm, acc_out,
               acc_sh, sall_v, dall_v, didx_v, rows_v, sem):
    c = lax.axis_index("c")
    s = lax.axis_index("s")
    wid = s * NC + c
    pltpu.sync_copy(zeros_hbm.at[pl.ds(s * RPT, RPT)],
                    acc_sh.at[pl.ds(s * RPT, RPT)])
    pltpu.sync_copy(src_hbm.at[pl.ds(wid * EPW, EPW)], sall_v)
    pltpu.sync_copy(dst_hbm.at[pl.ds(wid * EPW, EPW)], dall_v)
    plsc.subcore_barrier()

    @pl.loop(0, NCHUNK)
    def _chunk(j):
        for i in range(K // 16):
            didx_v[pl.ds(i * 16, 16)] = dall_v[pl.ds(j * K + i * 16, 16)]
        pltpu.async_copy(y_hbm.at[sall_v.at[pl.ds(j * K, K)]],
                         rows_v, sem).wait()
        pltpu.sync_copy(rows_v, acc_sh.at[didx_v], add=True)

    plsc.subcore_barrier()
    pltpu.sync_copy(acc_sh.at[pl.ds(s * RPT, RPT)],
                    acc_out.at[c, pl.ds(s * RPT, RPT)])


def _edge_pass(src, dst, y, zeros):
    return pl.kernel(
        _edge_body,
        out_type=jax.ShapeDtypeStruct((NC, N_PAD, DIM), jnp.float32),
        mesh=_MESH,
        scratch_types=[
            pltpu.VMEM_SHARED((N_PAD, DIM), jnp.float32),
            pltpu.VMEM((EPW,), jnp.int32),
            pltpu.VMEM((EPW,), jnp.int32),
            pltpu.VMEM((K,), jnp.int32),
            pltpu.VMEM((K, DIM), jnp.float32),
            pltpu.SemaphoreType.DMA,
        ],
    )(src, dst, y, zeros)


RB = 2000


def _scale_body(deg_ref, emb_ref, w_ref, y_ref, dinv_ref):
    deg = deg_ref[0, :, 0:1] + deg_ref[1, :, 0:1] + 1.0
    dinv = lax.rsqrt(jnp.maximum(deg, 1.0))
    xw = jnp.dot(emb_ref[...], w_ref[...], preferred_element_type=jnp.float32)
    y_ref[...] = xw * dinv
    dinv_ref[...] = dinv


def _scale(dega, emb, w):
    grid = N_NODES // RB
    return pl.pallas_call(
        _scale_body,
        grid=(grid,),
        in_specs=[
            pl.BlockSpec((NC, RB, DEGW), lambda i: (0, i, 0)),
            pl.BlockSpec((RB, DIM), lambda i: (i, 0)),
            pl.BlockSpec((DIM, DIM), lambda i: (0, 0)),
        ],
        out_specs=[
            pl.BlockSpec((RB, DIM), lambda i: (i, 0)),
            pl.BlockSpec((RB, 1), lambda i: (i, 0)),
        ],
        out_shape=[
            jax.ShapeDtypeStruct((N_PAD, DIM), jnp.float32),
            jax.ShapeDtypeStruct((N_NODES, 1), jnp.float32),
        ],
    )(dega, emb, w)


def _final_body(acc_ref, y_ref, dinv_ref, b_ref, out_ref):
    tot = acc_ref[0] + acc_ref[1] + y_ref[...]
    out_ref[...] = dinv_ref[...] * tot + b_ref[...]


def _final(acc, y, dinv, b2):
    grid = N_NODES // RB
    return pl.pallas_call(
        _final_body,
        grid=(grid,),
        in_specs=[
            pl.BlockSpec((NC, RB, DIM), lambda i: (0, i, 0)),
            pl.BlockSpec((RB, DIM), lambda i: (i, 0)),
            pl.BlockSpec((RB, 1), lambda i: (i, 0)),
            pl.BlockSpec((1, DIM), lambda i: (0, 0)),
        ],
        out_specs=pl.BlockSpec((RB, DIM), lambda i: (i, 0)),
        out_shape=jax.ShapeDtypeStruct((N_NODES, DIM), jnp.float32),
    )(acc, y, dinv, b2)


@jax.jit
def kernel(edge_index, emb, W, b):
    ei = jnp.asarray(edge_index, jnp.int32)
    src = ei[0]
    dst = ei[1]
    zeros = jnp.zeros((N_PAD, DIM), jnp.float32)

    dega = _degree_hist(dst)
    y, dinv = _scale(dega, emb, W)
    acc = _edge_pass(src, dst, y, zeros)
    out = _final(acc, y, dinv, b.reshape(1, DIM))
    return out
